# baseline (speedup 1.0000x reference)
import functools

import jax
import jax.numpy as jnp
from jax.experimental import pallas as pl

_INTERPRET = False

HID = 768
E = 8
T = 196
S = T + 1
_HI = jax.lax.Precision.HIGHEST


def _bf16r(x):
    return x.astype(jnp.bfloat16).astype(jnp.float32)


def _mixed_mm(xb, w):
    w_hi = _bf16r(w)
    w_lo = w - w_hi
    hi = jnp.dot(xb, w_hi, preferred_element_type=jnp.float32, precision=_HI)
    lo = jnp.dot(xb, w_lo, preferred_element_type=jnp.float32, precision=_HI)
    return hi + lo


def _proj_body(cls_ref, tok_ref, wc_ref, bc_ref, wt_ref, bt_ref, seq_ref):
    c = jnp.dot(cls_ref[0], wc_ref[...], preferred_element_type=jnp.float32,
                precision=_HI) + bc_ref[...]
    tb = _bf16r(tok_ref[0])
    t = _mixed_mm(tb, wt_ref[...]) + bt_ref[...]
    seq_ref[0, 0:1, :] = c
    seq_ref[0, 1:, :] = t


def _proj_seq(cls_in, tok_in, W_cls, b_cls, W_tok, b_tok):
    N = cls_in.shape[0]
    Dc = cls_in.shape[-1]
    Dt = tok_in.shape[-1]
    cls3 = cls_in.reshape(N, 1, Dc)
    return pl.pallas_call(
        _proj_body,
        grid=(N,),
        in_specs=[
            pl.BlockSpec((1, 1, Dc), lambda i: (i, 0, 0)),
            pl.BlockSpec((1, T, Dt), lambda i: (i, 0, 0)),
            pl.BlockSpec((Dc, HID), lambda i: (0, 0)),
            pl.BlockSpec((1, HID), lambda i: (0, 0)),
            pl.BlockSpec((Dt, HID), lambda i: (0, 0)),
            pl.BlockSpec((1, HID), lambda i: (0, 0)),
        ],
        out_specs=pl.BlockSpec((1, S, HID), lambda i: (i, 0, 0)),
        out_shape=jax.ShapeDtypeStruct((N, S, HID), jnp.float32),
        interpret=_INTERPRET,
    )(cls3, tok_in, W_cls, b_cls.reshape(1, HID), W_tok, b_tok.reshape(1, HID))


def _routing_comb(y, W_router, b_router):
    yf = y.reshape(-1, y.shape[-1])
    logits = jax.lax.dot_general(
        yf.astype(jnp.bfloat16), W_router, (((1,), (0,)), ((), ())),
        preferred_element_type=jnp.float32) + b_router
    probs = jax.nn.softmax(logits, axis=-1)
    topv, topi = jax.lax.top_k(probs, 2)
    w = topv / jnp.sum(topv, axis=-1, keepdims=True)
    mask = jax.nn.one_hot(topi, E, dtype=jnp.float32)
    comb = jnp.einsum('tk,tke->te', w, mask)
    return comb.reshape(y.shape[0], y.shape[1], E)


def _moe_body(seq_ref, comb_ref, we_ref, be_ref, out_ref):
    xb = _bf16r(seq_ref[0])
    cb = _bf16r(comb_ref[0])
    acc = jnp.zeros((S, HID), jnp.float32)
    for e in range(E):
        eo = _mixed_mm(xb, we_ref[e]) + be_ref[e:e + 1, :]
        acc = acc + cb[:, e:e + 1] * _bf16r(eo)
    out_ref[0] = acc


def _moe(seq, comb, W_exp, b_exp):
    N = seq.shape[0]
    return pl.pallas_call(
        _moe_body,
        grid=(N,),
        in_specs=[
            pl.BlockSpec((1, S, HID), lambda i: (i, 0, 0)),
            pl.BlockSpec((1, S, E), lambda i: (i, 0, 0)),
            pl.BlockSpec((E, HID, HID), lambda i: (0, 0, 0)),
            pl.BlockSpec((E, HID), lambda i: (0, 0)),
        ],
        out_specs=pl.BlockSpec((1, S, HID), lambda i: (i, 0, 0)),
        out_shape=jax.ShapeDtypeStruct((N, S, HID), jnp.float32),
        interpret=_INTERPRET,
    )(seq, comb, W_exp, b_exp)


def _attn_body(seq_ref, moe_ref, wg_ref, bg_ref, g_ref, b_ref, out_ref, *, gate_from_ctx):
    q = _bf16r(moe_ref[0, 0:1, :])
    kv = _bf16r(moe_ref[0, 1:, :])
    s = jnp.dot(q, kv.T, preferred_element_type=jnp.float32, precision=_HI)
    s = s - jnp.max(s, axis=-1, keepdims=True)
    p = jnp.exp(s)
    p = p / jnp.sum(p, axis=-1, keepdims=True)
    ctx = jnp.dot(_bf16r(p), kv, preferred_element_type=jnp.float32,
                  precision=_HI)
    orig = seq_ref[0, 0:1, :]
    gsrc = ctx if gate_from_ctx else orig
    g = jnp.tanh(jnp.dot(_bf16r(gsrc), wg_ref[...],
                         preferred_element_type=jnp.float32, precision=_HI)
                 + bg_ref[...])
    y = orig * g + ctx
    mu = jnp.mean(y, axis=-1, keepdims=True)
    var = jnp.mean((y - mu) ** 2, axis=-1, keepdims=True)
    out_ref[0] = (y - mu) / jnp.sqrt(var + 1e-5) * g_ref[...] + b_ref[...]


def _attn(seq, moe_out, W_gate, b_gate, ln_g, ln_b, gate_from_ctx):
    N = seq.shape[0]
    out = pl.pallas_call(
        functools.partial(_attn_body, gate_from_ctx=gate_from_ctx),
        grid=(N,),
        in_specs=[
            pl.BlockSpec((1, S, HID), lambda i: (i, 0, 0)),
            pl.BlockSpec((1, S, HID), lambda i: (i, 0, 0)),
            pl.BlockSpec((HID, 1), lambda i: (0, 0)),
            pl.BlockSpec((1, 1), lambda i: (0, 0)),
            pl.BlockSpec((1, HID), lambda i: (0, 0)),
            pl.BlockSpec((1, HID), lambda i: (0, 0)),
        ],
        out_specs=pl.BlockSpec((1, 1, HID), lambda i: (i, 0, 0)),
        out_shape=jax.ShapeDtypeStruct((N, 1, HID), jnp.float32),
        interpret=_INTERPRET,
    )(seq, moe_out, W_gate, b_gate.reshape(1, 1), ln_g.reshape(1, HID),
      ln_b.reshape(1, HID))
    return out.reshape(N, HID)


def _score_body(mti_ref, eti_ref, mit_ref, eit_ref, out_ref):
    a = jnp.dot(mti_ref[...], eti_ref[...].T, preferred_element_type=jnp.float32, precision=_HI)
    b = jnp.dot(mit_ref[...], eit_ref[...].T, preferred_element_type=jnp.float32, precision=_HI)
    out_ref[...] = (a + b) * 0.5


def _score(m_ti, e_ti, m_it, e_it):
    B_, NE_ = m_ti.shape[0], e_ti.shape[0]
    return pl.pallas_call(
        _score_body,
        out_shape=jax.ShapeDtypeStruct((B_, NE_), jnp.float32),
        interpret=_INTERPRET,
    )(m_ti, e_ti, m_it, e_it)


def kernel(entity_text_cls, entity_text_tokens, mention_text_cls, mention_text_tokens,
           entity_image_cls, entity_image_tokens, mention_image_cls, mention_image_tokens,
           W_text, b_text, W_img, b_img, W_gate, b_gate, ln_g, ln_b,
           W_router, b_router, W_exp, b_exp):
    A_seq = _proj_seq(entity_text_cls, entity_image_tokens,
                      W_text, b_text, W_img, b_img)
    B_seq = _proj_seq(mention_text_cls, mention_image_tokens,
                      W_text, b_text, W_img, b_img)
    C_seq = _proj_seq(entity_image_cls, entity_text_tokens,
                      W_img, b_img, W_text, b_text)
    D_seq = _proj_seq(mention_image_cls, mention_text_tokens,
                      W_img, b_img, W_text, b_text)

    _dn3 = (((2,), (0,)), ((), ()))

    def _mixed_tok_proj(tok, W, b):
        return jax.lax.dot_general(tok.astype(jnp.bfloat16), W, _dn3,
                                   preferred_element_type=jnp.float32) + b

    etc = entity_text_cls @ W_text + b_text
    mtc = mention_text_cls @ W_text + b_text
    eic = entity_image_cls @ W_img + b_img
    mic = mention_image_cls @ W_img + b_img
    ett = _mixed_tok_proj(entity_text_tokens, W_text, b_text)
    mtt = _mixed_tok_proj(mention_text_tokens, W_text, b_text)
    eit = _mixed_tok_proj(entity_image_tokens, W_img, b_img)
    mit = _mixed_tok_proj(mention_image_tokens, W_img, b_img)
    y_A = jnp.concatenate([etc[:, None, :], eit], axis=1)
    y_C = jnp.concatenate([eic[:, None, :], ett], axis=1)
    y_B = jnp.concatenate([mtc[:, None, :], mit], axis=1)
    y_D = jnp.concatenate([mic[:, None, :], mtt], axis=1)

    comb_A = _routing_comb(y_C, W_router, b_router)
    comb_B = _routing_comb(y_D, W_router, b_router)
    comb_C = _routing_comb(y_A, W_router, b_router)
    comb_D = _routing_comb(y_B, W_router, b_router)

    moeA = _moe(A_seq, comb_A, W_exp, b_exp)
    moeB = _moe(B_seq, comb_B, W_exp, b_exp)
    moeC = _moe(C_seq, comb_C, W_exp, b_exp)
    moeD = _moe(D_seq, comb_D, W_exp, b_exp)

    e_ti = _attn(A_seq, moeA, W_gate, b_gate, ln_g, ln_b, gate_from_ctx=True)
    m_ti = _attn(B_seq, moeB, W_gate, b_gate, ln_g, ln_b, gate_from_ctx=False)
    e_it = _attn(C_seq, moeC, W_gate, b_gate, ln_g, ln_b, gate_from_ctx=True)
    m_it = _attn(D_seq, moeD, W_gate, b_gate, ln_g, ln_b, gate_from_ctx=False)

    return _score(m_ti, e_ti, m_it, e_it)
